# Initial kernel scaffold; baseline (speedup 1.0000x reference)
import jax
import jax.numpy as jnp
from jax.experimental import pallas as pl


def kernel(x, edge_index, W1, b1, W2, b2, Wa, ba, Wb, bb):
    raise NotImplementedError("write your pallas kernel here")


# speedup vs baseline: 2.7227x; 2.7227x over previous
import functools

import jax
import jax.numpy as jnp
from jax import lax
from jax.experimental import pallas as pl
from jax.experimental.pallas import tpu as pltpu
from jax.experimental.pallas import tpu_sc as plsc

N = 10000
E = 320000
D = 128

NC = 2
NS = 16
NW = NC * NS
CHUNK = 128
EPW = 10240
E_PAD = NW * EPW
N_ACC = 10240
ZROWS = N_ACC // NS
ROW_BLK = 1000


def _segment_sum_sc(y, src_p, dst_p):
    mesh = plsc.VectorSubcoreMesh(
        core_axis_name="c", subcore_axis_name="s", num_cores=NC, num_subcores=NS
    )

    @functools.partial(
        pl.kernel,
        out_type=jax.ShapeDtypeStruct((NC, N_ACC, D), jnp.float32),
        mesh=mesh,
        scratch_types=[
            pltpu.VMEM((CHUNK,), jnp.int32),
            pltpu.VMEM((CHUNK,), jnp.int32),
            pltpu.VMEM((CHUNK, D), jnp.float32),
            pltpu.VMEM_SHARED((N_ACC, D), jnp.float32),
            pltpu.SemaphoreType.DMA,
        ],
    )
    def seg_kernel(y_hbm, src_hbm, dst_hbm, out_hbm, src_v, dst_v, rows_v, acc, sem):
        c = lax.axis_index("c")
        s = lax.axis_index("s")
        wid = c * NS + s

        zero = jnp.zeros((16,), jnp.float32)

        def zero_row(r, carry):
            for k in range(D // 16):
                rows_v[r, pl.ds(k * 16, 16)] = zero
            return carry

        lax.fori_loop(0, CHUNK, zero_row, 0)
        zbase = s * ZROWS
        for j in range(ZROWS // CHUNK):
            pltpu.sync_copy(rows_v, acc.at[pl.ds(zbase + j * CHUNK, CHUNK)])
        plsc.subcore_barrier()

        ebase = wid * EPW

        def body(k, carry):
            off = pl.multiple_of(ebase + k * CHUNK, 8)
            pltpu.sync_copy(src_hbm.at[pl.ds(off, CHUNK)], src_v)
            pltpu.sync_copy(dst_hbm.at[pl.ds(off, CHUNK)], dst_v)
            pltpu.async_copy(y_hbm.at[src_v], rows_v, sem).wait()
            pltpu.sync_copy(rows_v, acc.at[dst_v], add=True)
            return carry

        lax.fori_loop(0, EPW // CHUNK, body, 0)
        plsc.subcore_barrier()

        pltpu.sync_copy(acc.at[pl.ds(zbase, ZROWS)], out_hbm.at[c, pl.ds(zbase, ZROWS)])

    return seg_kernel(y, src_p, dst_p)


def _mm_kernel(x_ref, w_ref, o_ref):
    o_ref[:] = jnp.dot(
        x_ref[:], w_ref[:], preferred_element_type=jnp.float32,
        precision=lax.Precision.HIGHEST,
    )


def _matmul_tc(x, w):
    k = w.shape[1]
    return pl.pallas_call(
        _mm_kernel,
        grid=(N // ROW_BLK,),
        in_specs=[
            pl.BlockSpec((ROW_BLK, D), lambda i: (i, 0)),
            pl.BlockSpec((D, k), lambda i: (0, 0)),
        ],
        out_specs=pl.BlockSpec((ROW_BLK, k), lambda i: (i, 0)),
        out_shape=jax.ShapeDtypeStruct((N, k), jnp.float32),
    )(x, w)


def _fuse_kernel(p_ref, b_ref, w_ref, bo_ref, o_ref):
    h = jnp.maximum(p_ref[0] + p_ref[1] + b_ref[:], 0.0)
    o_ref[:] = (
        jnp.dot(h, w_ref[:], preferred_element_type=jnp.float32,
                precision=lax.Precision.HIGHEST)
        + bo_ref[:]
    )


def _fused_relu_mm_tc(p, b, w, b_out):
    k = w.shape[1]
    return pl.pallas_call(
        _fuse_kernel,
        grid=(N // ROW_BLK,),
        in_specs=[
            pl.BlockSpec((NC, ROW_BLK, D), lambda i: (0, i, 0)),
            pl.BlockSpec((1, D), lambda i: (0, 0)),
            pl.BlockSpec((D, k), lambda i: (0, 0)),
            pl.BlockSpec((1, k), lambda i: (0, 0)),
        ],
        out_specs=pl.BlockSpec((ROW_BLK, k), lambda i: (i, 0)),
        out_shape=jax.ShapeDtypeStruct((N, k), jnp.float32),
    )(p, b, w, b_out)


def kernel(x, edge_index, W1, b1, W2, b2, Wa, ba, Wb, bb):
    src = edge_index[0].astype(jnp.int32)
    dst = edge_index[1].astype(jnp.int32)
    pad = E_PAD - E
    src_p = jnp.concatenate([src, jnp.zeros((pad,), jnp.int32)])
    dst_p = jnp.concatenate([dst, jnp.full((pad,), N, jnp.int32)])

    KAB = 24
    Wab = jnp.zeros((D, KAB), jnp.float32)
    Wab = Wab.at[:, :2].set(Wa).at[:, 2:18].set(Wb)
    bab = jnp.zeros((1, KAB), jnp.float32).at[0, :2].set(ba).at[0, 2:18].set(bb)

    y1 = _matmul_tc(x, W1)
    p1 = _segment_sum_sc(y1, src_p, dst_p)
    y2 = _fused_relu_mm_tc(p1, b1.reshape(1, D), W2, jnp.zeros((1, D), jnp.float32))
    p2 = _segment_sum_sc(y2, src_p, dst_p)
    xab = _fused_relu_mm_tc(p2, b2.reshape(1, D), Wab, bab)
    return (xab[:, :2], xab[:, 2:18])


# speedup vs baseline: 3.4806x; 1.2784x over previous
import functools

import jax
import jax.numpy as jnp
from jax import lax
from jax.experimental import pallas as pl
from jax.experimental.pallas import tpu as pltpu
from jax.experimental.pallas import tpu_sc as plsc

N = 10000
E = 320000
D = 128

NC = 2
NS = 16
NW = NC * NS
CHUNK = 128
EPW = 10240
E_PAD = NW * EPW
N_ACC = 10240
ZROWS = N_ACC // NS
ROW_BLK = 1000


NCH = EPW // CHUNK
NG = NCH // 4


def _segment_sum_sc(y, src_p, dst_p):
    mesh = plsc.VectorSubcoreMesh(
        core_axis_name="c", subcore_axis_name="s", num_cores=NC, num_subcores=NS
    )

    @functools.partial(
        pl.kernel,
        out_type=jax.ShapeDtypeStruct((NC, N_ACC, D), jnp.float32),
        mesh=mesh,
        scratch_types=[
            pltpu.VMEM_SHARED((N_ACC, D), jnp.float32),
        ],
    )
    def seg_kernel(y_hbm, src_hbm, dst_hbm, out_hbm, acc):
      def scoped(rows0, rows1, si0, si1, si2, si3, di0, di1, di2, di3,
                 i0, i1, i2, i3, ga, gb):
        rows = (rows0, rows1)
        src_i = (si0, si1, si2, si3)
        dst_i = (di0, di1, di2, di3)
        isem = (i0, i1, i2, i3)
        gsem = (ga, gb)
        c = lax.axis_index("c")
        s = lax.axis_index("s")
        wid = c * NS + s
        cb = wid * NCH

        zero = jnp.zeros((16,), jnp.float32)

        def zero_row(r, carry):
            for k in range(D // 16):
                rows0[r, pl.ds(k * 16, 16)] = zero
            return carry

        lax.fori_loop(0, CHUNK, zero_row, 0)
        zbase = s * ZROWS
        for j in range(ZROWS // CHUNK):
            pltpu.sync_copy(rows0, acc.at[pl.ds(zbase + j * CHUNK, CHUNK)])
        plsc.subcore_barrier()

        def eoff(k):
            return pl.multiple_of((cb + k) * CHUNK, 8)

        def idx_start(k, q):
            pltpu.async_copy(src_hbm.at[pl.ds(eoff(k), CHUNK)], src_i[q], isem[q])
            pltpu.async_copy(dst_hbm.at[pl.ds(eoff(k), CHUNK)], dst_i[q], isem[q])

        def idx_wait(k, q):
            pltpu.make_async_copy(src_hbm.at[pl.ds(eoff(k), CHUNK)], src_i[q], isem[q]).wait()
            pltpu.make_async_copy(dst_hbm.at[pl.ds(eoff(k), CHUNK)], dst_i[q], isem[q]).wait()

        def gather_start(q, b):
            pltpu.async_copy(y_hbm.at[src_i[q]], rows[b], gsem[b])

        def gather_wait(q, b):
            pltpu.make_async_copy(y_hbm.at[src_i[q]], rows[b], gsem[b]).wait()

        def step(k, b, *, last_group):
            if not (last_group and b >= 2):
                idx_start(k + 2, (b + 2) % 4)
            if not (last_group and b == 3):
                idx_wait(k + 1, (b + 1) % 4)
                gather_start((b + 1) % 4, (b + 1) % 2)
            gather_wait(b, b % 2)
            pltpu.sync_copy(rows[b % 2], acc.at[dst_i[b]], add=True)

        idx_start(0, 0)
        idx_start(1, 1)
        idx_wait(0, 0)
        gather_start(0, 0)
        for b in range(4):
            step(b, b, last_group=False)

        def body(j, carry):
            k0 = j * 4
            for b in range(4):
                step(k0 + b, b, last_group=False)
            return carry

        lax.fori_loop(1, NG - 1, body, 0)

        for b in range(4):
            step((NG - 1) * 4 + b, b, last_group=True)
        plsc.subcore_barrier()

        pltpu.sync_copy(acc.at[pl.ds(zbase, ZROWS)], out_hbm.at[c, pl.ds(zbase, ZROWS)])

      pl.run_scoped(
          scoped,
          pltpu.VMEM((CHUNK, D), jnp.float32),
          pltpu.VMEM((CHUNK, D), jnp.float32),
          *[pltpu.VMEM((CHUNK,), jnp.int32) for _ in range(8)],
          *[pltpu.SemaphoreType.DMA for _ in range(6)],
      )

    return seg_kernel(y, src_p, dst_p)


def _mm_kernel(x_ref, w_ref, o_ref):
    o_ref[:] = jnp.dot(
        x_ref[:], w_ref[:], preferred_element_type=jnp.float32,
        precision=lax.Precision.HIGHEST,
    )


def _matmul_tc(x, w):
    k = w.shape[1]
    return pl.pallas_call(
        _mm_kernel,
        grid=(N // ROW_BLK,),
        in_specs=[
            pl.BlockSpec((ROW_BLK, D), lambda i: (i, 0)),
            pl.BlockSpec((D, k), lambda i: (0, 0)),
        ],
        out_specs=pl.BlockSpec((ROW_BLK, k), lambda i: (i, 0)),
        out_shape=jax.ShapeDtypeStruct((N, k), jnp.float32),
    )(x, w)


def _fuse_kernel(p_ref, b_ref, w_ref, bo_ref, o_ref):
    h = jnp.maximum(p_ref[0] + p_ref[1] + b_ref[:], 0.0)
    o_ref[:] = (
        jnp.dot(h, w_ref[:], preferred_element_type=jnp.float32,
                precision=lax.Precision.HIGHEST)
        + bo_ref[:]
    )


def _fused_relu_mm_tc(p, b, w, b_out):
    k = w.shape[1]
    return pl.pallas_call(
        _fuse_kernel,
        grid=(N // ROW_BLK,),
        in_specs=[
            pl.BlockSpec((NC, ROW_BLK, D), lambda i: (0, i, 0)),
            pl.BlockSpec((1, D), lambda i: (0, 0)),
            pl.BlockSpec((D, k), lambda i: (0, 0)),
            pl.BlockSpec((1, k), lambda i: (0, 0)),
        ],
        out_specs=pl.BlockSpec((ROW_BLK, k), lambda i: (i, 0)),
        out_shape=jax.ShapeDtypeStruct((N, k), jnp.float32),
    )(p, b, w, b_out)


def kernel(x, edge_index, W1, b1, W2, b2, Wa, ba, Wb, bb):
    src = edge_index[0].astype(jnp.int32)
    dst = edge_index[1].astype(jnp.int32)
    pad = E_PAD - E
    src_p = jnp.concatenate([src, jnp.zeros((pad,), jnp.int32)])
    dst_p = jnp.concatenate([dst, jnp.full((pad,), N, jnp.int32)])

    KAB = 24
    Wab = jnp.zeros((D, KAB), jnp.float32)
    Wab = Wab.at[:, :2].set(Wa).at[:, 2:18].set(Wb)
    bab = jnp.zeros((1, KAB), jnp.float32).at[0, :2].set(ba).at[0, 2:18].set(bb)

    y1 = _matmul_tc(x, W1)
    p1 = _segment_sum_sc(y1, src_p, dst_p)
    y2 = _fused_relu_mm_tc(p1, b1.reshape(1, D), W2, jnp.zeros((1, D), jnp.float32))
    p2 = _segment_sum_sc(y2, src_p, dst_p)
    xab = _fused_relu_mm_tc(p2, b2.reshape(1, D), Wab, bab)
    return (xab[:, :2], xab[:, 2:18])
